# Initial kernel scaffold; baseline (speedup 1.0000x reference)
import jax
import jax.numpy as jnp
from jax.experimental import pallas as pl


def kernel(x, edge_index, W1_l, b1, W1_r, W2_l, b2, W2_r, W_head, b_head):
    raise NotImplementedError("write your pallas kernel here")


# speedup vs baseline: 4.0563x; 4.0563x over previous
import functools

import jax
import jax.numpy as jnp
from jax import lax
from jax.experimental import pallas as pl
from jax.experimental.pallas import tpu as pltpu
from jax.experimental.pallas import tpu_sc as plsc

N = 10000
E = 320000
D = 128
NC = 2
NS = 16
NW = NC * NS
CHUNK = 128
PT = ((E + NW - 1) // NW + CHUNK - 1) // CHUNK * CHUNK
EPAD = PT * NW
NCHUNK = PT // CHUNK
NPAD = 10240
STRIPE = NPAD // NS

_mesh = plsc.VectorSubcoreMesh(core_axis_name="c", subcore_axis_name="s",
                               num_cores=NC, num_subcores=NS)


def _zero_fill(buf, nrows, ncols):
    z16 = jnp.zeros((16,), jnp.float32)

    def fill(i, _):
        buf[i // (ncols // 16), pl.ds((i % (ncols // 16)) * 16, 16)] = z16
        return 0

    lax.fori_loop(0, nrows * (ncols // 16), fill, 0)


def _seg_core(src_hbm, dst_hbm, feat_hbm, out_sum, sum_sh, sidx, didx, rows):
    c = lax.axis_index("c")
    s = lax.axis_index("s")

    _zero_fill(rows, CHUNK, D)
    row0 = s * STRIPE

    def zcopy(j, _):
        pltpu.sync_copy(rows, sum_sh.at[pl.ds(row0 + j * CHUNK, CHUNK), :])
        return 0

    lax.fori_loop(0, STRIPE // CHUNK, zcopy, 0)
    plsc.subcore_barrier()

    g = c * NS + s
    ebase = g * PT

    def echunk(i, _):
        base = ebase + i * CHUNK
        pltpu.sync_copy(src_hbm.at[pl.ds(base, CHUNK)], sidx)
        pltpu.sync_copy(dst_hbm.at[pl.ds(base, CHUNK)], didx)
        pltpu.sync_copy(feat_hbm.at[sidx], rows)
        pltpu.sync_copy(rows, sum_sh.at[didx], add=True)
        return 0

    lax.fori_loop(0, NCHUNK, echunk, 0)
    plsc.subcore_barrier()

    pltpu.sync_copy(sum_sh.at[pl.ds(row0, STRIPE), :],
                    out_sum.at[c, pl.ds(row0, STRIPE), :])


@functools.partial(
    pl.kernel,
    out_type=jax.ShapeDtypeStruct((NW, NPAD), jnp.float32),
    mesh=_mesh,
    scratch_types=dict(
        didx=pltpu.VMEM((CHUNK,), jnp.int32),
        cntv=pltpu.VMEM((NPAD,), jnp.float32),
    ),
    compiler_params=pltpu.CompilerParams(needs_layout_passes=False),
)
def _degree(dst_hbm, out_cnt, didx, cntv):
    c = lax.axis_index("c")
    s = lax.axis_index("s")
    z16 = jnp.zeros((16,), jnp.float32)

    def czf(i, _):
        cntv[pl.ds(i * 16, 16)] = z16
        return 0

    lax.fori_loop(0, NPAD // 16, czf, 0)
    g = c * NS + s
    ebase = g * PT
    one16 = jnp.ones((16,), jnp.float32)

    def echunk(i, _):
        pltpu.sync_copy(dst_hbm.at[pl.ds(ebase + i * CHUNK, CHUNK)], didx)

        def cadd(j, _):
            v = didx[pl.ds(j * 16, 16)]
            plsc.addupdate_scatter(cntv, [v], one16)
            return 0

        lax.fori_loop(0, CHUNK // 16, cadd, 0)
        return 0

    lax.fori_loop(0, NCHUNK, echunk, 0)
    pltpu.sync_copy(cntv, out_cnt.at[g])


@functools.partial(
    pl.kernel,
    out_type=jax.ShapeDtypeStruct((NC, NPAD, D), jnp.float32),
    mesh=_mesh,
    scratch_types=dict(
        sidx=pltpu.VMEM((CHUNK,), jnp.int32),
        didx=pltpu.VMEM((CHUNK,), jnp.int32),
        rows=pltpu.VMEM((CHUNK, D), jnp.float32),
        sum_sh=pltpu.VMEM_SHARED((NPAD, D), jnp.float32),
    ),
)
def _seg_sum(src_hbm, dst_hbm, feat_hbm, out_sum,
             sidx, didx, rows, sum_sh):
    _seg_core(src_hbm, dst_hbm, feat_hbm, out_sum, sum_sh, sidx, didx, rows)


BN = 1024


def _layer1_body(p0, p1, cn, x, wl, wr, b, out):
    cnt = jnp.maximum(jnp.sum(cn[...], axis=0), 1.0)
    mean = (p0[...] + p1[...]) / cnt[:, None]
    acc = jnp.dot(mean, wl[...], preferred_element_type=jnp.float32)
    acc = acc + jnp.dot(x[...], wr[...], preferred_element_type=jnp.float32)
    out[...] = jnp.maximum(acc + b[...], 0.0)


def _layer2_body(q0, q1, cn, h, wl, wr, b, wh, bh, out):
    cnt = jnp.maximum(jnp.sum(cn[...], axis=0), 1.0)
    mean = (q0[...] + q1[...]) / cnt[:, None]
    acc = jnp.dot(mean, wl[...], preferred_element_type=jnp.float32)
    acc = acc + jnp.dot(h[...], wr[...], preferred_element_type=jnp.float32)
    h2 = jnp.maximum(acc + b[...], 0.0)
    out[...] = jnp.dot(h2, wh[...], preferred_element_type=jnp.float32) + bh[...]


def _row_spec(w):
    return pl.BlockSpec((BN, w), lambda i: (i, 0))


def _cnt_spec():
    return pl.BlockSpec((NW, BN), lambda i: (0, i))


def _full_spec(r, cdim):
    return pl.BlockSpec((r, cdim), lambda i: (0, 0))


_layer1 = pl.pallas_call(
    _layer1_body,
    grid=(NPAD // BN,),
    in_specs=[_row_spec(D), _row_spec(D), _cnt_spec(),
              _row_spec(D), _full_spec(D, D), _full_spec(D, D), _full_spec(1, D)],
    out_specs=_row_spec(D),
    out_shape=jax.ShapeDtypeStruct((NPAD, D), jnp.float32),
)

_layer2 = pl.pallas_call(
    _layer2_body,
    grid=(NPAD // BN,),
    in_specs=[_row_spec(D), _row_spec(D), _cnt_spec(),
              _row_spec(D), _full_spec(D, D), _full_spec(D, D), _full_spec(1, D),
              _full_spec(D, D), _full_spec(1, D)],
    out_specs=_row_spec(D),
    out_shape=jax.ShapeDtypeStruct((NPAD, D), jnp.float32),
)


def kernel(x, edge_index, W1_l, b1, W1_r, W2_l, b2, W2_r, W_head, b_head):
    src = edge_index[0]
    dst = edge_index[1]
    pad = EPAD - E
    src_p = jnp.concatenate([src, jnp.zeros((pad,), jnp.int32)])
    dst_p = jnp.concatenate([dst, jnp.full((pad,), N, jnp.int32)])
    x_p = jnp.zeros((NPAD, D), jnp.float32).at[:N].set(x)

    cnts = _degree(dst_p)
    sums1 = _seg_sum(src_p, dst_p, x_p)
    h1 = _layer1(sums1[0], sums1[1], cnts, x_p, W1_l, W1_r, b1.reshape(1, D))
    sums2 = _seg_sum(src_p, dst_p, h1)
    wh = jnp.zeros((D, D), jnp.float32).at[:, :3].set(W_head)
    bh = jnp.zeros((1, D), jnp.float32).at[0, :3].set(b_head)
    out = _layer2(sums2[0], sums2[1], cnts, h1, W2_l, W2_r, b2.reshape(1, D),
                  wh, bh)
    return out[:N, :3]
